# baseline (speedup 1.0000x reference)
import functools

import jax
import jax.numpy as jnp
from jax import lax
from jax.experimental import pallas as pl
from jax.experimental.pallas import tpu as pltpu
from jax.experimental.pallas import tpu_sc as plsc

_D_MODEL = 128
_D_TIME = 32
_N_HW = 200
_N_TOD = 1440
_N_ROWS = _N_HW + _N_TOD
_N_ROWS_PAD = 1664
_B, _L = 4096, 200
_N = _B * _L


def _silu(x):
    return x / (1.0 + jnp.exp(-x))


def _lut_body(hour_t, wd_t, tw1v, tb1, tw2t, tb2,
              dw1v, db1, dw2t, db2, pwt, pb, out_ref):
    p1t = pwt[0:32, :]
    p2t = pwt[32:64, :]
    p3t = pwt[64:96, :]
    p4t = pwt[96:128, :]

    j = lax.broadcasted_iota(jnp.int32, (_N_HW, 1), 0)
    h_idx = lax.div(j, 8)
    w_idx = lax.rem(j, 8)
    oh_h = (h_idx == lax.broadcasted_iota(jnp.int32, (_N_HW, 25), 1)).astype(jnp.float32)
    oh_w = (w_idx == lax.broadcasted_iota(jnp.int32, (_N_HW, 8), 1)).astype(jnp.float32)
    hour_rows = jnp.dot(oh_h, hour_t[...], preferred_element_type=jnp.float32)
    wd_rows = jnp.dot(oh_w, wd_t[...], preferred_element_type=jnp.float32)

    dow_c = w_idx.astype(jnp.float32) / 7.0
    dh = _silu(dow_c * dw1v[...] + db1[...])
    dow_enc = jnp.dot(dh, dw2t[...], preferred_element_type=jnp.float32) + db2[...]

    hw = (jnp.dot(hour_rows, p1t, preferred_element_type=jnp.float32)
          + jnp.dot(wd_rows, p2t, preferred_element_type=jnp.float32)
          + jnp.dot(dow_enc, p4t, preferred_element_type=jnp.float32))
    out_ref[0:_N_HW, :] = hw

    m = lax.broadcasted_iota(jnp.int32, (_N_TOD, 1), 0).astype(jnp.float32) / 1440.0
    th = _silu(m * tw1v[...] + tb1[...])
    tod_enc = jnp.dot(th, tw2t[...], preferred_element_type=jnp.float32) + tb2[...]
    tod = jnp.dot(tod_enc, p3t, preferred_element_type=jnp.float32) + pb[...]
    out_ref[_N_HW:_N_ROWS, :] = tod
    out_ref[_N_ROWS:_N_ROWS_PAD, :] = jnp.zeros((_N_ROWS_PAD - _N_ROWS, _D_MODEL), jnp.float32)


def _build_lut(hour_table, weekday_table, tod_w1, tod_b1, tod_w2, tod_b2,
               dow_w1, dow_b1, dow_w2, dow_b2, proj_w, proj_b, *, interpret=False):
    return pl.pallas_call(
        _lut_body,
        out_shape=jax.ShapeDtypeStruct((_N_ROWS_PAD, _D_MODEL), jnp.float32),
        interpret=interpret,
    )(
        hour_table, weekday_table,
        tod_w1.reshape(1, _D_TIME), tod_b1.reshape(1, _D_TIME),
        tod_w2.T, tod_b2.reshape(1, _D_TIME),
        dow_w1.reshape(1, _D_TIME), dow_b1.reshape(1, _D_TIME),
        dow_w2.T, dow_b2.reshape(1, _D_TIME),
        proj_w.T, proj_b.reshape(1, _D_MODEL),
    )


_CHUNK = 128
_NW = 32
_PER_W = _N // _NW
_NCHUNK = _PER_W // _CHUNK


def _sc_gather(table, hwm_packed):
    info = plsc.get_sparse_core_info()
    nc = info.num_cores
    ns = info.num_subcores
    mesh = plsc.VectorSubcoreMesh(core_axis_name="c", subcore_axis_name="s")

    rows_per_tile = _N_ROWS_PAD // ns

    idxv = pltpu.VMEM((_CHUNK,), jnp.int32)
    hwmv = pltpu.VMEM((3, _CHUNK), jnp.int32)
    rowv = pltpu.VMEM((_CHUNK, _D_MODEL), jnp.float32)
    sharedv = pltpu.VMEM_SHARED((_N_ROWS_PAD, _D_MODEL), jnp.float32)
    sem = pltpu.SemaphoreType.DMA

    @functools.partial(
        pl.kernel,
        mesh=mesh,
        out_type=jax.ShapeDtypeStruct((_N, _D_MODEL), jnp.float32),
        scratch_types=([sharedv] + [hwmv] * 2 + [idxv] * 4 + [rowv] * 4 + [rowv] * 2
                       + [sem] * 2 + [sem] * 4 + [sem] * 2 + [sem] * 4),
    )
    def run(table_hbm, hwm_hbm, out_hbm,
            shared,
            hwm0, hwm1,
            i1_0, i1_1, i2_0, i2_1,
            a0, a1, a2, a3, b0, b1,
            shwm0, shwm1, sa0, sa1, sa2, sa3, sbm0, sbm1,
            so0, so1, so2, so3):
        hwm = (hwm0, hwm1)
        i1 = (i1_0, i1_1)
        i2 = (i2_0, i2_1)
        ba = (a0, a1, a2, a3)
        bb = (b0, b1)
        shwm = (shwm0, shwm1)
        sa = (sa0, sa1, sa2, sa3)
        sbm = (sbm0, sbm1)
        so = (so0, so1, so2, so3)

        wid = lax.axis_index("s") * nc + lax.axis_index("c")
        sid = lax.axis_index("s")
        base0 = wid * _PER_W
        cid0 = wid * _NCHUNK

        srow = sid * rows_per_tile
        pltpu.sync_copy(table_hbm.at[pl.ds(srow, rows_per_tile)],
                        shared.at[pl.ds(srow, rows_per_tile)])
        plsc.subcore_barrier()

        def issue_idx(g, q):
            pltpu.async_copy(hwm_hbm.at[cid0 + g], hwm[q], shwm[q])

        def wait_idx(q):
            pltpu.make_async_copy(hwm_hbm.at[0], hwm[q], shwm[q]).wait()

        def compute_idx(q):
            for j in range(_CHUNK // 16):
                sl = pl.ds(j * 16, 16)
                i1[q][sl] = hwm[q][0, sl] * 8 + hwm[q][1, sl]
                i2[q][sl] = hwm[q][2, sl] + _N_HW

        def issue_gather(q, r):
            pltpu.async_copy(shared.at[i1[q]], ba[r], sa[r])
            pltpu.async_copy(shared.at[i2[q]], bb[q], sbm[q])

        def wait_gather(q, r):
            pltpu.make_async_copy(shared.at[i1[q]], ba[r], sa[r]).wait()
            pltpu.make_async_copy(shared.at[i2[q]], bb[q], sbm[q]).wait()

        def wait_out(r):
            pltpu.make_async_copy(
                ba[r], out_hbm.at[pl.ds(base0, _CHUNK)], so[r]).wait()

        def add_rows(q, r):
            def body(r4, c):
                for rr in range(4):
                    row = r4 * 4 + rr
                    for cj in range(_D_MODEL // 16):
                        cs = pl.ds(cj * 16, 16)
                        plsc.addupdate(ba[r].at[row, cs], bb[q][row, cs])
                return c
            lax.fori_loop(0, _CHUNK // 4, body, 0)

        def store_out(g, r):
            base = base0 + g * _CHUNK
            pltpu.async_copy(ba[r], out_hbm.at[pl.ds(base, _CHUNK)], so[r])

        issue_idx(0, 0)
        wait_idx(0)
        compute_idx(0)
        issue_gather(0, 0)
        issue_idx(1, 1)

        def super_body(s, carry):
            for b4 in (0, 1, 2, 3):
                g = 4 * s + b4
                q = b4 % 2
                q1 = (b4 + 1) % 2
                r = b4
                r1 = (b4 + 1) % 4

                @pl.when(g + 1 < _NCHUNK)
                def _():
                    wait_idx(q1)
                    compute_idx(q1)

                    @pl.when(g >= 3)
                    def _():
                        wait_out(r1)

                    issue_gather(q1, r1)

                @pl.when(g + 2 < _NCHUNK)
                def _():
                    issue_idx(g + 2, q)

                wait_gather(q, r)
                add_rows(q, r)
                store_out(g, r)
            return carry

        lax.fori_loop(0, _NCHUNK // 4, super_body, 0)
        for r in range(4):
            wait_out(r)

    return run(table, hwm_packed)


def kernel(hours, weekdays, start_mins, hour_table, weekday_table,
           tod_w1, tod_b1, tod_w2, tod_b2,
           dow_w1, dow_b1, dow_w2, dow_b2,
           proj_w, proj_b):
    table = _build_lut(hour_table, weekday_table, tod_w1, tod_b1, tod_w2,
                       tod_b2, dow_w1, dow_b1, dow_w2, dow_b2, proj_w, proj_b)
    hwm_packed = jnp.stack(
        [hours.reshape(_N // _CHUNK, _CHUNK).astype(jnp.int32),
         weekdays.reshape(_N // _CHUNK, _CHUNK).astype(jnp.int32),
         start_mins.reshape(_N // _CHUNK, _CHUNK).astype(jnp.int32)],
        axis=1)
    out = _sc_gather(table, hwm_packed)
    return out.reshape(_B, _L, _D_MODEL)


# speedup vs baseline: 1.4989x; 1.4989x over previous
import functools

import jax
import jax.numpy as jnp
from jax import lax
from jax.experimental import pallas as pl
from jax.experimental.pallas import tpu as pltpu
from jax.experimental.pallas import tpu_sc as plsc

_D_MODEL = 128
_D_TIME = 32
_N_HW = 200
_N_TOD = 1440
_N_ROWS = _N_HW + _N_TOD
_N_ROWS_PAD = 1792
_B, _L = 4096, 200
_N = _B * _L


def _silu(x):
    return x / (1.0 + jnp.exp(-x))


def _lut_body(hour_t, wd_t, tw1v, tb1, tw2t, tb2,
              dw1v, db1, dw2t, db2, pwt, pb, out_ref, bf_ref):
    p1t = pwt[0:32, :]
    p2t = pwt[32:64, :]
    p3t = pwt[64:96, :]
    p4t = pwt[96:128, :]

    j = lax.broadcasted_iota(jnp.int32, (_N_HW, 1), 0)
    h_idx = lax.div(j, 8)
    w_idx = lax.rem(j, 8)
    oh_h = (h_idx == lax.broadcasted_iota(jnp.int32, (_N_HW, 25), 1)).astype(jnp.float32)
    oh_w = (w_idx == lax.broadcasted_iota(jnp.int32, (_N_HW, 8), 1)).astype(jnp.float32)
    hour_rows = jnp.dot(oh_h, hour_t[...], preferred_element_type=jnp.float32)
    wd_rows = jnp.dot(oh_w, wd_t[...], preferred_element_type=jnp.float32)

    dow_c = w_idx.astype(jnp.float32) / 7.0
    dh = _silu(dow_c * dw1v[...] + db1[...])
    dow_enc = jnp.dot(dh, dw2t[...], preferred_element_type=jnp.float32) + db2[...]

    hw = (jnp.dot(hour_rows, p1t, preferred_element_type=jnp.float32)
          + jnp.dot(wd_rows, p2t, preferred_element_type=jnp.float32)
          + jnp.dot(dow_enc, p4t, preferred_element_type=jnp.float32))
    out_ref[0:_N_HW, :] = hw

    m = lax.broadcasted_iota(jnp.int32, (_N_TOD, 1), 0).astype(jnp.float32) / 1440.0
    th = _silu(m * tw1v[...] + tb1[...])
    tod_enc = jnp.dot(th, tw2t[...], preferred_element_type=jnp.float32) + tb2[...]
    tod = jnp.dot(tod_enc, p3t, preferred_element_type=jnp.float32) + pb[...]
    out_ref[_N_HW:_N_ROWS, :] = tod
    out_ref[_N_ROWS:_N_ROWS_PAD, :] = jnp.zeros((_N_ROWS_PAD - _N_ROWS, _D_MODEL), jnp.float32)
    bf_ref[...] = out_ref[...].astype(jnp.bfloat16)


def _build_lut(hour_table, weekday_table, tod_w1, tod_b1, tod_w2, tod_b2,
               dow_w1, dow_b1, dow_w2, dow_b2, proj_w, proj_b, *, interpret=False):
    return pl.pallas_call(
        _lut_body,
        out_shape=(jax.ShapeDtypeStruct((_N_ROWS_PAD, _D_MODEL), jnp.float32),
                   jax.ShapeDtypeStruct((_N_ROWS_PAD, _D_MODEL), jnp.bfloat16)),
        interpret=interpret,
    )(
        hour_table, weekday_table,
        tod_w1.reshape(1, _D_TIME), tod_b1.reshape(1, _D_TIME),
        tod_w2.T, tod_b2.reshape(1, _D_TIME),
        dow_w1.reshape(1, _D_TIME), dow_b1.reshape(1, _D_TIME),
        dow_w2.T, dow_b2.reshape(1, _D_TIME),
        proj_w.T, proj_b.reshape(1, _D_MODEL),
    )


_CHUNK = 128
_NW = 32
_PER_W = _N // _NW
_NCHUNK = _PER_W // _CHUNK


def _sc_gather(table_bf, hwm_packed):
    info = plsc.get_sparse_core_info()
    nc = info.num_cores
    ns = info.num_subcores
    mesh = plsc.VectorSubcoreMesh(core_axis_name="c", subcore_axis_name="s")

    rows_per_tile = _N_ROWS_PAD // ns

    idxv = pltpu.VMEM((_CHUNK,), jnp.int32)
    hwmv = pltpu.VMEM((3, _CHUNK), jnp.int32)
    rowbf = pltpu.VMEM((_CHUNK, _D_MODEL // 2), jnp.int32)
    rowf = pltpu.VMEM((_CHUNK, _D_MODEL), jnp.float32)
    sharedv = pltpu.VMEM_SHARED((_N_ROWS_PAD, _D_MODEL // 2), jnp.int32)
    sem = pltpu.SemaphoreType.DMA

    @functools.partial(
        pl.kernel,
        mesh=mesh,
        out_type=jax.ShapeDtypeStruct((_N, _D_MODEL), jnp.float32),
        scratch_types=([sharedv] + [hwmv] * 2 + [idxv] * 4 + [rowbf] * 4
                       + [rowf] * 2
                       + [sem] * 2 + [sem] * 2 + [sem] * 2 + [sem] * 2),
    )
    def run(table_hbm, hwm_hbm, out_hbm,
            shared,
            hwm0, hwm1,
            i1_0, i1_1, i2_0, i2_1,
            a0, a1, b0, b1,
            f0, f1,
            shwm0, shwm1, sa0, sa1, sbm0, sbm1,
            so0, so1):
        hwm = (hwm0, hwm1)
        i1 = (i1_0, i1_1)
        i2 = (i2_0, i2_1)
        ba = (a0, a1)
        bb = (b0, b1)
        fo = (f0, f1)
        shwm = (shwm0, shwm1)
        sa = (sa0, sa1)
        sbm = (sbm0, sbm1)
        so = (so0, so1)

        wid = lax.axis_index("s") * nc + lax.axis_index("c")
        sid = lax.axis_index("s")
        base0 = wid * _PER_W
        cid0 = wid * _NCHUNK

        srow = sid * rows_per_tile
        pltpu.sync_copy(table_hbm.at[pl.ds(srow, rows_per_tile)],
                        shared.at[pl.ds(srow, rows_per_tile)])
        plsc.subcore_barrier()

        def issue_idx(g, q):
            pltpu.async_copy(hwm_hbm.at[cid0 + g], hwm[q], shwm[q])

        def wait_idx(q):
            pltpu.make_async_copy(hwm_hbm.at[0], hwm[q], shwm[q]).wait()

        def compute_idx(q):
            for j in range(_CHUNK // 16):
                sl = pl.ds(j * 16, 16)
                i1[q][sl] = hwm[q][0, sl] * 8 + hwm[q][1, sl]
                i2[q][sl] = hwm[q][2, sl] + _N_HW

        def issue_gather(q):
            pltpu.async_copy(shared.at[i1[q]], ba[q], sa[q])
            pltpu.async_copy(shared.at[i2[q]], bb[q], sbm[q])

        def wait_gather(q):
            pltpu.make_async_copy(shared.at[i1[q]], ba[q], sa[q]).wait()
            pltpu.make_async_copy(shared.at[i2[q]], bb[q], sbm[q]).wait()

        def wait_out(r):
            pltpu.make_async_copy(
                fo[r], out_hbm.at[pl.ds(base0, _CHUNK)], so[r]).wait()

        def merge_rows(q, r):
            def body(r4, c):
                hi_mask = jnp.full((16,), -65536, jnp.int32)
                for rr in range(4):
                    row = r4 * 4 + rr
                    for gg in range(_D_MODEL // 32):
                        va = ba[q][row, pl.ds(gg * 16, 16)]
                        vb = bb[q][row, pl.ds(gg * 16, 16)]
                        a_lo = lax.bitcast_convert_type(va << 16, jnp.float32)
                        b_lo = lax.bitcast_convert_type(vb << 16, jnp.float32)
                        a_hi = lax.bitcast_convert_type(va & hi_mask, jnp.float32)
                        b_hi = lax.bitcast_convert_type(vb & hi_mask, jnp.float32)
                        fo[r][row, pl.ds(gg * 32, 16)] = a_lo + b_lo
                        fo[r][row, pl.ds(gg * 32 + 16, 16)] = a_hi + b_hi
                return c
            lax.fori_loop(0, _CHUNK // 4, body, 0)

        def store_out(g, r):
            base = base0 + g * _CHUNK
            pltpu.async_copy(fo[r], out_hbm.at[pl.ds(base, _CHUNK)], so[r])

        issue_idx(0, 0)
        wait_idx(0)
        compute_idx(0)
        issue_gather(0)
        issue_idx(1, 1)

        def super_body(s, carry):
            for b4 in (0, 1, 2, 3):
                g = 4 * s + b4
                q = b4 % 2
                q1 = (b4 + 1) % 2

                @pl.when(g + 1 < _NCHUNK)
                def _():
                    wait_idx(q1)
                    compute_idx(q1)
                    issue_gather(q1)

                @pl.when(g + 2 < _NCHUNK)
                def _():
                    issue_idx(g + 2, q)

                @pl.when(g >= 2)
                def _():
                    wait_out(q)

                wait_gather(q)
                merge_rows(q, q)
                store_out(g, q)
            return carry

        lax.fori_loop(0, _NCHUNK // 4, super_body, 0)
        for r in range(2):
            wait_out(r)

    return run(table_bf, hwm_packed)


def kernel(hours, weekdays, start_mins, hour_table, weekday_table,
           tod_w1, tod_b1, tod_w2, tod_b2,
           dow_w1, dow_b1, dow_w2, dow_b2,
           proj_w, proj_b):
    _, lut_bf = _build_lut(hour_table, weekday_table, tod_w1, tod_b1, tod_w2,
                           tod_b2, dow_w1, dow_b1, dow_w2, dow_b2, proj_w,
                           proj_b)
    t = lut_bf.reshape(_N_ROWS_PAD, _D_MODEL // 32, 2, 16)
    table_perm = jnp.swapaxes(t, 2, 3).reshape(_N_ROWS_PAD, _D_MODEL // 2, 2)
    table_bf = lax.bitcast_convert_type(table_perm, jnp.int32)
    hwm_packed = jnp.stack(
        [hours.reshape(_N // _CHUNK, _CHUNK).astype(jnp.int32),
         weekdays.reshape(_N // _CHUNK, _CHUNK).astype(jnp.int32),
         start_mins.reshape(_N // _CHUNK, _CHUNK).astype(jnp.int32)],
        axis=1)
    out = _sc_gather(table_bf, hwm_packed)
    return out.reshape(_B, _L, _D_MODEL)


# speedup vs baseline: 1.5008x; 1.0013x over previous
import functools

import jax
import jax.numpy as jnp
from jax import lax
from jax.experimental import pallas as pl
from jax.experimental.pallas import tpu as pltpu
from jax.experimental.pallas import tpu_sc as plsc

_D_MODEL = 128
_D_TIME = 32
_N_HW = 200
_N_TOD = 1440
_N_ROWS = _N_HW + _N_TOD
_N_ROWS_PAD = 1792
_B, _L = 4096, 200
_N = _B * _L


def _silu(x):
    return x / (1.0 + jnp.exp(-x))


def _lut_body(hour_t, wd_t, tw1v, tb1, tw2t, tb2,
              dw1v, db1, dw2t, db2, pwt, pb, out_ref, bf_ref):
    p1t = pwt[0:32, :]
    p2t = pwt[32:64, :]
    p3t = pwt[64:96, :]
    p4t = pwt[96:128, :]

    j = lax.broadcasted_iota(jnp.int32, (_N_HW, 1), 0)
    h_idx = lax.div(j, 8)
    w_idx = lax.rem(j, 8)
    oh_h = (h_idx == lax.broadcasted_iota(jnp.int32, (_N_HW, 25), 1)).astype(jnp.float32)
    oh_w = (w_idx == lax.broadcasted_iota(jnp.int32, (_N_HW, 8), 1)).astype(jnp.float32)
    hour_rows = jnp.dot(oh_h, hour_t[...], preferred_element_type=jnp.float32)
    wd_rows = jnp.dot(oh_w, wd_t[...], preferred_element_type=jnp.float32)

    dow_c = w_idx.astype(jnp.float32) / 7.0
    dh = _silu(dow_c * dw1v[...] + db1[...])
    dow_enc = jnp.dot(dh, dw2t[...], preferred_element_type=jnp.float32) + db2[...]

    hw = (jnp.dot(hour_rows, p1t, preferred_element_type=jnp.float32)
          + jnp.dot(wd_rows, p2t, preferred_element_type=jnp.float32)
          + jnp.dot(dow_enc, p4t, preferred_element_type=jnp.float32))
    out_ref[0:_N_HW, :] = hw

    m = lax.broadcasted_iota(jnp.int32, (_N_TOD, 1), 0).astype(jnp.float32) / 1440.0
    th = _silu(m * tw1v[...] + tb1[...])
    tod_enc = jnp.dot(th, tw2t[...], preferred_element_type=jnp.float32) + tb2[...]
    tod = jnp.dot(tod_enc, p3t, preferred_element_type=jnp.float32) + pb[...]
    out_ref[_N_HW:_N_ROWS, :] = tod
    out_ref[_N_ROWS:_N_ROWS_PAD, :] = jnp.zeros((_N_ROWS_PAD - _N_ROWS, _D_MODEL), jnp.float32)
    bf_ref[...] = out_ref[...].astype(jnp.bfloat16)


def _build_lut(hour_table, weekday_table, tod_w1, tod_b1, tod_w2, tod_b2,
               dow_w1, dow_b1, dow_w2, dow_b2, proj_w, proj_b, *, interpret=False):
    return pl.pallas_call(
        _lut_body,
        out_shape=(jax.ShapeDtypeStruct((_N_ROWS_PAD, _D_MODEL), jnp.float32),
                   jax.ShapeDtypeStruct((_N_ROWS_PAD, _D_MODEL), jnp.bfloat16)),
        interpret=interpret,
    )(
        hour_table, weekday_table,
        tod_w1.reshape(1, _D_TIME), tod_b1.reshape(1, _D_TIME),
        tod_w2.T, tod_b2.reshape(1, _D_TIME),
        dow_w1.reshape(1, _D_TIME), dow_b1.reshape(1, _D_TIME),
        dow_w2.T, dow_b2.reshape(1, _D_TIME),
        proj_w.T, proj_b.reshape(1, _D_MODEL),
    )


_CHUNK = 128
_NW = 32
_PER_W = _N // _NW
_NCHUNK = _PER_W // _CHUNK


def _sc_gather(table_bf, hwm_packed):
    info = plsc.get_sparse_core_info()
    nc = info.num_cores
    ns = info.num_subcores
    mesh = plsc.VectorSubcoreMesh(core_axis_name="c", subcore_axis_name="s")

    rows_per_tile = _N_ROWS_PAD // ns

    idxv = pltpu.VMEM((_CHUNK,), jnp.int32)
    hwmv = pltpu.VMEM((3, _CHUNK), jnp.int32)
    rowbf = pltpu.VMEM((_CHUNK, _D_MODEL // 2), jnp.int32)
    rowf = pltpu.VMEM((_CHUNK, _D_MODEL), jnp.float32)
    sharedv = pltpu.VMEM_SHARED((_N_ROWS_PAD, _D_MODEL // 2), jnp.int32)
    sem = pltpu.SemaphoreType.DMA

    @functools.partial(
        pl.kernel,
        mesh=mesh,
        out_type=jax.ShapeDtypeStruct((_N, _D_MODEL), jnp.float32),
        scratch_types=([sharedv] + [hwmv] * 2 + [idxv] * 4 + [rowbf] * 4
                       + [rowf] * 2
                       + [sem] * 2 + [sem] * 2 + [sem] * 2 + [sem] * 2),
    )
    def run(table_hbm, hwm_hbm, out_hbm,
            shared,
            hwm0, hwm1,
            i1_0, i1_1, i2_0, i2_1,
            a0, a1, b0, b1,
            f0, f1,
            shwm0, shwm1, sa0, sa1, sbm0, sbm1,
            so0, so1):
        hwm = (hwm0, hwm1)
        i1 = (i1_0, i1_1)
        i2 = (i2_0, i2_1)
        ba = (a0, a1)
        bb = (b0, b1)
        fo = (f0, f1)
        shwm = (shwm0, shwm1)
        sa = (sa0, sa1)
        sbm = (sbm0, sbm1)
        so = (so0, so1)

        wid = lax.axis_index("s") * nc + lax.axis_index("c")
        sid = lax.axis_index("s")
        base0 = wid * _PER_W
        cid0 = wid * _NCHUNK

        srow = sid * rows_per_tile
        pltpu.sync_copy(table_hbm.at[pl.ds(srow, rows_per_tile)],
                        shared.at[pl.ds(srow, rows_per_tile)])
        plsc.subcore_barrier()

        def issue_idx(g, q):
            pltpu.async_copy(hwm_hbm.at[cid0 + g], hwm[q], shwm[q])

        def wait_idx(q):
            pltpu.make_async_copy(hwm_hbm.at[0], hwm[q], shwm[q]).wait()

        def compute_idx(q):
            for j in range(_CHUNK // 16):
                sl = pl.ds(j * 16, 16)
                i1[q][sl] = hwm[q][0, sl] * 8 + hwm[q][1, sl]
                i2[q][sl] = hwm[q][2, sl] + _N_HW

        def issue_gather(q):
            pltpu.async_copy(shared.at[i1[q]], ba[q], sa[q])
            pltpu.async_copy(shared.at[i2[q]], bb[q], sbm[q])

        def wait_gather(q):
            pltpu.make_async_copy(shared.at[i1[q]], ba[q], sa[q]).wait()
            pltpu.make_async_copy(shared.at[i2[q]], bb[q], sbm[q]).wait()

        def wait_out(r):
            pltpu.make_async_copy(
                fo[r], out_hbm.at[pl.ds(base0, _CHUNK)], so[r]).wait()

        def merge_rows(q, r):
            def body(r4, c):
                hi_mask = jnp.full((16,), -65536, jnp.int32)
                for rr in range(4):
                    row = r4 * 4 + rr
                    for gg in range(_D_MODEL // 32):
                        va = ba[q][row, pl.ds(gg * 16, 16)]
                        vb = bb[q][row, pl.ds(gg * 16, 16)]
                        a_lo = lax.bitcast_convert_type(va << 16, jnp.float32)
                        b_lo = lax.bitcast_convert_type(vb << 16, jnp.float32)
                        a_hi = lax.bitcast_convert_type(va & hi_mask, jnp.float32)
                        b_hi = lax.bitcast_convert_type(vb & hi_mask, jnp.float32)
                        fo[r][row, pl.ds(gg * 32, 16)] = a_hi + b_hi
                        fo[r][row, pl.ds(gg * 32 + 16, 16)] = a_lo + b_lo
                return c
            lax.fori_loop(0, _CHUNK // 4, body, 0)

        def store_out(g, r):
            base = base0 + g * _CHUNK
            pltpu.async_copy(fo[r], out_hbm.at[pl.ds(base, _CHUNK)], so[r])

        issue_idx(0, 0)
        wait_idx(0)
        compute_idx(0)
        issue_gather(0)
        issue_idx(1, 1)

        def super_body(s, carry):
            for b4 in (0, 1, 2, 3):
                g = 4 * s + b4
                q = b4 % 2
                q1 = (b4 + 1) % 2

                @pl.when(g + 1 < _NCHUNK)
                def _():
                    wait_idx(q1)
                    compute_idx(q1)
                    issue_gather(q1)

                @pl.when(g + 2 < _NCHUNK)
                def _():
                    issue_idx(g + 2, q)

                @pl.when(g >= 2)
                def _():
                    wait_out(q)

                wait_gather(q)
                merge_rows(q, q)
                store_out(g, q)
            return carry

        lax.fori_loop(0, _NCHUNK // 4, super_body, 0)
        for r in range(2):
            wait_out(r)

    return run(table_bf, hwm_packed)


def kernel(hours, weekdays, start_mins, hour_table, weekday_table,
           tod_w1, tod_b1, tod_w2, tod_b2,
           dow_w1, dow_b1, dow_w2, dow_b2,
           proj_w, proj_b):
    _, lut_bf = _build_lut(hour_table, weekday_table, tod_w1, tod_b1, tod_w2,
                           tod_b2, dow_w1, dow_b1, dow_w2, dow_b2, proj_w,
                           proj_b)
    t = lut_bf.reshape(_N_ROWS_PAD, _D_MODEL // 32, 2, 16)
    table_perm = jnp.swapaxes(t, 2, 3).reshape(_N_ROWS_PAD, _D_MODEL // 2, 2)
    table_bf = lax.bitcast_convert_type(table_perm, jnp.int32)
    hwm_packed = jnp.stack(
        [hours.reshape(_N // _CHUNK, _CHUNK).astype(jnp.int32),
         weekdays.reshape(_N // _CHUNK, _CHUNK).astype(jnp.int32),
         start_mins.reshape(_N // _CHUNK, _CHUNK).astype(jnp.int32)],
        axis=1)
    out = _sc_gather(table_bf, hwm_packed)
    return out.reshape(_B, _L, _D_MODEL)
